# Initial kernel scaffold; baseline (speedup 1.0000x reference)
import jax
import jax.numpy as jnp
from jax.experimental import pallas as pl


def kernel(x, pos, edge_index, lattice, batch, W_lin, W_src, W_dst, W_pos, b_pos, W_lat, b_lat, W1, b1, W2, b2, W3, b3, W_out, b_out):
    raise NotImplementedError("write your pallas kernel here")


# speedup vs baseline: 4.9395x; 4.9395x over previous
import functools

import jax
import jax.numpy as jnp
from jax import lax
from jax.experimental import pallas as pl
from jax.experimental.pallas import tpu as pltpu
from jax.experimental.pallas import tpu_sc as plsc

N = 10000
E = 320000
HID = 128
NB = 64
HH = HID // 2

NR = 10240
TRASH = N
NTILES = 16
EPT = E // NTILES
K = 80
NCHUNK = EPT // K
RPT = NR // NTILES
BN = 1000
NG = N // BN


def _prep_body(x_ref, p_ref, wl_ref, ws_ref, wp_ref, bp_ref, pdt_ref, qvt_ref):
    xb = x_ref[...]
    pb = p_ref[...]
    P = jnp.dot(pb, wp_ref[...], preferred_element_type=jnp.float32)
    x_lin = jnp.dot(xb, wl_ref[...], preferred_element_type=jnp.float32)
    a_src = jnp.dot(xb, ws_ref[...], preferred_element_type=jnp.float32)
    bp = bp_ref[...]
    Pd = P + bp
    Qs = P + a_src
    Vs = x_lin - P
    pdt_ref[...] = Pd
    qvt_ref[0, :, :] = jnp.concatenate([Qs[:, :HH], Vs[:, :HH]], axis=1)
    qvt_ref[1, :, :] = jnp.concatenate([Qs[:, HH:], Vs[:, HH:]], axis=1)


def _prep_call(x, posp, W_lin, W_src, wposp, bp2):
    full = lambda s: pl.BlockSpec(s, lambda i: (0, 0))
    return pl.pallas_call(
        _prep_body,
        grid=(NG,),
        in_specs=[
            pl.BlockSpec((BN, HID), lambda i: (i, 0)),
            pl.BlockSpec((BN, 8), lambda i: (i, 0)),
            full((HID, HID)),
            full((HID, HID)),
            full((8, HID)),
            full((1, HID)),
        ],
        out_specs=[
            pl.BlockSpec((BN, HID), lambda i: (i, 0)),
            pl.BlockSpec((2, BN, HID), lambda i: (0, i, 0)),
        ],
        out_shape=[
            jax.ShapeDtypeStruct((N, HID), jnp.float32),
            jax.ShapeDtypeStruct((2, N, HID), jnp.float32),
        ],
    )(x, posp, W_lin, W_src, wposp, bp2)


def _edge_body(src_hbm, dst_hbm, pdt_hbm, qvt_hbm, out_hbm,
               s_idx, d_idx, s_adj, d_scat,
               pd_buf, qv_buf, val_buf, acc, sem):
    h = lax.axis_index("c")
    sid = lax.axis_index("s")
    hN = h * N
    hoff = h * HH

    def zbody(j, _):
        for q in range(HID // 16):
            val_buf[j, pl.ds(q * 16, 16)] = jnp.zeros((16,), jnp.float32)
        return 0
    lax.fori_loop(0, K, zbody, 0)
    for k in range(RPT // K):
        pltpu.sync_copy(val_buf, acc.at[pl.ds(sid * RPT + k * K, K)])
    plsc.subcore_barrier()

    ebase = sid * EPT

    def chunk_body(c, _):
        base = ebase + c * K
        pltpu.sync_copy(src_hbm.at[pl.ds(base, K)], s_idx)
        pltpu.sync_copy(dst_hbm.at[pl.ds(base, K)], d_idx)
        for j in range(K // 16):
            sl = pl.ds(j * 16, 16)
            sv = s_idx[sl]
            dv = d_idx[sl]
            off = jnp.zeros((16,), jnp.int32) + hN
            s_adj[sl] = sv + off
            d_scat[sl] = jnp.where(sv == dv,
                                   jnp.full((16,), TRASH, jnp.int32), dv)
        pltpu.async_copy(pdt_hbm.at[d_idx], pd_buf, sem).wait()
        pltpu.async_copy(qvt_hbm.at[s_adj], qv_buf, sem).wait()

        def ebody(j, _):
            for q in range(HH // 16):
                sl = pl.ds(q * 16, 16)
                sl2 = pl.ds(HH + q * 16, 16)
                pd = pd_buf[j, pl.ds(hoff + q * 16, 16)]
                qs = qv_buf[j, sl]
                vs = qv_buf[j, sl2]
                ex = jnp.exp(pd - qs)
                val_buf[j, sl] = ex
                val_buf[j, sl2] = ex * (vs + pd)
            return 0
        lax.fori_loop(0, K, ebody, 0)
        pltpu.sync_copy(val_buf, acc.at[d_scat], add=True)
        return 0

    lax.fori_loop(0, NCHUNK, chunk_body, 0)
    plsc.subcore_barrier()
    pltpu.sync_copy(acc.at[pl.ds(sid * RPT, RPT)],
                    out_hbm.at[pl.ds(h * NR + sid * RPT, RPT)])


@functools.cache
def _edge_call():
    return pl.kernel(
        _edge_body,
        out_type=jax.ShapeDtypeStruct((2 * NR, HID), jnp.float32),
        mesh=plsc.VectorSubcoreMesh(core_axis_name="c", subcore_axis_name="s"),
        scratch_types=[
            pltpu.VMEM((K,), jnp.int32),
            pltpu.VMEM((K,), jnp.int32),
            pltpu.VMEM((K,), jnp.int32),
            pltpu.VMEM((K,), jnp.int32),
            pltpu.VMEM((K, HID), jnp.float32),
            pltpu.VMEM((K, HID), jnp.float32),
            pltpu.VMEM((K, HID), jnp.float32),
            pltpu.VMEM_SHARED((NR, HID), jnp.float32),
            pltpu.SemaphoreType.DMA,
        ],
    )


def _finish_body(acc_ref, x_ref, b_ref, ws_ref, wl_ref, bp_ref,
                 lat_ref, wlat_ref, blat_ref,
                 w1_ref, b1_ref, w2_ref, b2_ref, w3_ref, b3_ref,
                 wo_ref, bo_ref, out_ref, pooled_ref, cnt_ref):
    i = pl.program_id(0)
    xb = x_ref[...]
    a_src = jnp.dot(xb, ws_ref[...], preferred_element_type=jnp.float32)
    x_lin = jnp.dot(xb, wl_ref[...], preferred_element_type=jnp.float32)
    bp = bp_ref[...]
    es = jnp.exp(bp - a_src)
    a0 = acc_ref[0, :, :]
    a1 = acc_ref[1, :, :]
    den = jnp.concatenate([a0[:, :HH], a1[:, :HH]], axis=1) + es
    num = jnp.concatenate([a0[:, HH:], a1[:, HH:]], axis=1) + es * (x_lin + bp)
    nodes = num / (den + 1e-16)

    b = b_ref[...].reshape(1, BN)
    onehot = (b == lax.broadcasted_iota(jnp.int32, (NB, BN), 0))
    onehot = onehot.astype(jnp.float32)
    ps = jnp.dot(onehot, nodes, preferred_element_type=jnp.float32)
    cs = jnp.dot(onehot, jnp.ones((BN, HID), jnp.float32),
                 preferred_element_type=jnp.float32)

    @pl.when(i == 0)
    def _():
        pooled_ref[...] = ps
        cnt_ref[...] = cs

    @pl.when(i > 0)
    def _():
        pooled_ref[...] += ps
        cnt_ref[...] += cs

    @pl.when(i == NG - 1)
    def _():
        pooled = pooled_ref[...] / jnp.maximum(cnt_ref[...], 1.0)
        l = jnp.dot(lat_ref[...], wlat_ref[...],
                    preferred_element_type=jnp.float32) + blat_ref[...]
        hh = jnp.concatenate([pooled, l], axis=1)
        hh = jnp.maximum(jnp.dot(hh, w1_ref[...],
                                 preferred_element_type=jnp.float32)
                         + b1_ref[...], 0.0)
        hh = jnp.maximum(jnp.dot(hh, w2_ref[...],
                                 preferred_element_type=jnp.float32)
                         + b2_ref[...], 0.0)
        hh = jnp.maximum(jnp.dot(hh, w3_ref[...],
                                 preferred_element_type=jnp.float32)
                         + b3_ref[...], 0.0)
        out_ref[...] = jnp.dot(hh, wo_ref[...],
                               preferred_element_type=jnp.float32) + bo_ref[...]


def _finish_call(acc3, x, batch3, W_src, W_lin, bp2, latp, wlatp, blat2,
                 W1, b12, W2, b22, W3, b32, wop, bo2):
    full = lambda s: pl.BlockSpec(s, lambda i: tuple(0 for _ in s))
    return pl.pallas_call(
        _finish_body,
        grid=(NG,),
        in_specs=[
            pl.BlockSpec((2, BN, HID), lambda i: (0, i, 0)),
            pl.BlockSpec((BN, HID), lambda i: (i, 0)),
            pl.BlockSpec((1, 1, BN), lambda i: (i, 0, 0)),
            full((HID, HID)),
            full((HID, HID)),
            full((1, HID)),
            full((NB, 16)),
            full((16, HID)),
            full((1, HID)),
            full((2 * HID, 3 * HID)),
            full((1, 3 * HID)),
            full((3 * HID, 2 * HID)),
            full((1, 2 * HID)),
            full((2 * HID, HID)),
            full((1, HID)),
            full((HID, HID)),
            full((1, HID)),
        ],
        out_specs=pl.BlockSpec((NB, HID), lambda i: (0, 0)),
        out_shape=jax.ShapeDtypeStruct((NB, HID), jnp.float32),
        scratch_shapes=[
            pltpu.VMEM((NB, HID), jnp.float32),
            pltpu.VMEM((NB, HID), jnp.float32),
        ],
    )(acc3, x, batch3, W_src, W_lin, bp2, latp, wlatp, blat2,
      W1, b12, W2, b22, W3, b32, wop, bo2)


def kernel(x, pos, edge_index, lattice, batch, W_lin, W_src, W_dst, W_pos,
           b_pos, W_lat, b_lat, W1, b1, W2, b2, W3, b3, W_out, b_out):
    posp = jnp.pad(pos, ((0, 0), (0, 5)))
    wposp = jnp.pad(W_pos, ((0, 5), (0, 0)))
    bp2 = b_pos.reshape(1, HID)

    pdt, qvt = _prep_call(x, posp, W_lin, W_src, wposp, bp2)

    acc = _edge_call()(edge_index[0], edge_index[1],
                       pdt, qvt.reshape(2 * N, HID))

    latp = jnp.pad(lattice, ((0, 0), (0, 7)))
    wlatp = jnp.pad(W_lat, ((0, 7), (0, 0)))
    wop = jnp.pad(W_out, ((0, 0), (0, HID - 1)))
    bo2 = jnp.pad(b_out, (0, HID - 1)).reshape(1, HID)

    out = _finish_call(acc.reshape(2, NR, HID), x,
                       batch.reshape(NG, 1, BN),
                       W_src, W_lin, bp2,
                       latp, wlatp, b_lat.reshape(1, HID),
                       W1, b1.reshape(1, 3 * HID),
                       W2, b2.reshape(1, 2 * HID),
                       W3, b3.reshape(1, HID),
                       wop, bo2)
    return out[:, :1]


# speedup vs baseline: 6.3865x; 1.2929x over previous
import functools

import jax
import jax.numpy as jnp
from jax import lax
from jax.experimental import pallas as pl
from jax.experimental.pallas import tpu as pltpu
from jax.experimental.pallas import tpu_sc as plsc

N = 10000
E = 320000
HID = 128
NB = 64
HH = HID // 2

NR = 10240
TRASH = N
NTILES = 16
K = 80
NCHUNK = 250
EPT = NCHUNK * K
RPT = NR // NTILES
BN = 1000
NG = N // BN


def _prep_body(x_ref, p_ref, wl_ref, ws_ref, wp_ref, bp_ref, pdt_ref, qvt_ref):
    xb = x_ref[...]
    pb = p_ref[...]
    P = jnp.dot(pb, wp_ref[...], preferred_element_type=jnp.float32)
    x_lin = jnp.dot(xb, wl_ref[...], preferred_element_type=jnp.float32)
    a_src = jnp.dot(xb, ws_ref[...], preferred_element_type=jnp.float32)
    bp = bp_ref[...]
    Pd = P + bp
    Qs = P + a_src
    Vs = x_lin - P
    pdt_ref[...] = Pd
    qvt_ref[0, :, :] = jnp.concatenate([Qs[:, :HH], Vs[:, :HH]], axis=1)
    qvt_ref[1, :, :] = jnp.concatenate([Qs[:, HH:], Vs[:, HH:]], axis=1)


def _prep_call(x, posp, W_lin, W_src, wposp, bp2):
    full = lambda s: pl.BlockSpec(s, lambda i: (0, 0))
    return pl.pallas_call(
        _prep_body,
        grid=(NG,),
        in_specs=[
            pl.BlockSpec((BN, HID), lambda i: (i, 0)),
            pl.BlockSpec((BN, 8), lambda i: (i, 0)),
            full((HID, HID)),
            full((HID, HID)),
            full((8, HID)),
            full((1, HID)),
        ],
        out_specs=[
            pl.BlockSpec((BN, HID), lambda i: (i, 0)),
            pl.BlockSpec((2, BN, HID), lambda i: (0, i, 0)),
        ],
        out_shape=[
            jax.ShapeDtypeStruct((N, HID), jnp.float32),
            jax.ShapeDtypeStruct((2, N, HID), jnp.float32),
        ],
    )(x, posp, W_lin, W_src, wposp, bp2)


def _edge_body(src_hbm, dst_hbm, pdt_hbm, qvt_hbm, out_hbm,
               s0, s1, d0, d1, e0, e1, pd0, pd1, qv0, qv1,
               acc, sem0, sem1):
    h = lax.axis_index("c")
    sid = lax.axis_index("s")
    hN = h * N
    hoff = h * HH
    S = (s0, s1)
    D = (d0, d1)
    SCT = (e0, e1)
    PD = (pd0, pd1)
    QV = (qv0, qv1)
    SEM = (sem0, sem1)

    def zbody(j, _):
        for q in range(HID // 16):
            qv0[j, pl.ds(q * 16, 16)] = jnp.zeros((16,), jnp.float32)
        return 0
    lax.fori_loop(0, K, zbody, 0)
    for k in range(RPT // K):
        pltpu.sync_copy(qv0, acc.at[pl.ds(sid * RPT + k * K, K)])
    plsc.subcore_barrier()

    ebase = sid * EPT

    def issue(c, b):
        base = ebase + c * K
        pltpu.sync_copy(src_hbm.at[pl.ds(base, K)], S[b])
        pltpu.sync_copy(dst_hbm.at[pl.ds(base, K)], D[b])
        for j in range(K // 16):
            sl = pl.ds(j * 16, 16)
            sv = S[b][sl]
            dv = D[b][sl]
            off = jnp.zeros((16,), jnp.int32) + hN
            SCT[b][sl] = jnp.where(sv == dv,
                                   jnp.full((16,), TRASH, jnp.int32), dv)
            S[b][sl] = sv + off
        cp_pd = pltpu.async_copy(pdt_hbm.at[D[b]], PD[b], SEM[b])
        cp_qv = pltpu.async_copy(qvt_hbm.at[S[b]], QV[b], SEM[b])
        return cp_pd, cp_qv

    def process(cps, b):
        cps[0].wait()
        cps[1].wait()

        def ebody(j, _):
            for q in range(HH // 16):
                sl = pl.ds(q * 16, 16)
                sl2 = pl.ds(HH + q * 16, 16)
                pd = PD[b][j, pl.ds(hoff + q * 16, 16)]
                qs = QV[b][j, sl]
                vs = QV[b][j, sl2]
                ex = jnp.exp(pd - qs)
                QV[b][j, sl] = ex
                QV[b][j, sl2] = ex * (vs + pd)
            return 0
        lax.fori_loop(0, K, ebody, 0)
        pltpu.sync_copy(QV[b], acc.at[SCT[b]], add=True)

    def body(c2, _):
        cps0 = issue(c2 * 2, 0)
        cps1 = issue(c2 * 2 + 1, 1)
        process(cps0, 0)
        process(cps1, 1)
        return 0
    lax.fori_loop(0, NCHUNK // 2, body, 0)

    plsc.subcore_barrier()
    pltpu.sync_copy(acc.at[pl.ds(sid * RPT, RPT)],
                    out_hbm.at[pl.ds(h * NR + sid * RPT, RPT)])


@functools.cache
def _edge_call():
    return pl.kernel(
        _edge_body,
        out_type=jax.ShapeDtypeStruct((2 * NR, HID), jnp.float32),
        mesh=plsc.VectorSubcoreMesh(core_axis_name="c", subcore_axis_name="s"),
        scratch_types=[
            pltpu.VMEM((K,), jnp.int32),
            pltpu.VMEM((K,), jnp.int32),
            pltpu.VMEM((K,), jnp.int32),
            pltpu.VMEM((K,), jnp.int32),
            pltpu.VMEM((K,), jnp.int32),
            pltpu.VMEM((K,), jnp.int32),
            pltpu.VMEM((K, HID), jnp.float32),
            pltpu.VMEM((K, HID), jnp.float32),
            pltpu.VMEM((K, HID), jnp.float32),
            pltpu.VMEM((K, HID), jnp.float32),
            pltpu.VMEM_SHARED((NR, HID), jnp.float32),
            pltpu.SemaphoreType.DMA,
            pltpu.SemaphoreType.DMA,
        ],
    )


def _finish_body(acc_ref, x_ref, b_ref, ws_ref, wl_ref, bp_ref,
                 lat_ref, wlat_ref, blat_ref,
                 w1_ref, b1_ref, w2_ref, b2_ref, w3_ref, b3_ref,
                 wo_ref, bo_ref, out_ref, pooled_ref, cnt_ref):
    i = pl.program_id(0)
    xb = x_ref[...]
    a_src = jnp.dot(xb, ws_ref[...], preferred_element_type=jnp.float32)
    x_lin = jnp.dot(xb, wl_ref[...], preferred_element_type=jnp.float32)
    bp = bp_ref[...]
    es = jnp.exp(bp - a_src)
    a0 = acc_ref[0, :, :]
    a1 = acc_ref[1, :, :]
    den = jnp.concatenate([a0[:, :HH], a1[:, :HH]], axis=1) + es
    num = jnp.concatenate([a0[:, HH:], a1[:, HH:]], axis=1) + es * (x_lin + bp)
    nodes = num / (den + 1e-16)

    b = b_ref[...].reshape(1, BN)
    onehot = (b == lax.broadcasted_iota(jnp.int32, (NB, BN), 0))
    onehot = onehot.astype(jnp.float32)
    ps = jnp.dot(onehot, nodes, preferred_element_type=jnp.float32)
    cs = jnp.dot(onehot, jnp.ones((BN, HID), jnp.float32),
                 preferred_element_type=jnp.float32)

    @pl.when(i == 0)
    def _():
        pooled_ref[...] = ps
        cnt_ref[...] = cs

    @pl.when(i > 0)
    def _():
        pooled_ref[...] += ps
        cnt_ref[...] += cs

    @pl.when(i == NG - 1)
    def _():
        pooled = pooled_ref[...] / jnp.maximum(cnt_ref[...], 1.0)
        l = jnp.dot(lat_ref[...], wlat_ref[...],
                    preferred_element_type=jnp.float32) + blat_ref[...]
        hh = jnp.concatenate([pooled, l], axis=1)
        hh = jnp.maximum(jnp.dot(hh, w1_ref[...],
                                 preferred_element_type=jnp.float32)
                         + b1_ref[...], 0.0)
        hh = jnp.maximum(jnp.dot(hh, w2_ref[...],
                                 preferred_element_type=jnp.float32)
                         + b2_ref[...], 0.0)
        hh = jnp.maximum(jnp.dot(hh, w3_ref[...],
                                 preferred_element_type=jnp.float32)
                         + b3_ref[...], 0.0)
        out_ref[...] = jnp.dot(hh, wo_ref[...],
                               preferred_element_type=jnp.float32) + bo_ref[...]


def _finish_call(acc3, x, batch3, W_src, W_lin, bp2, latp, wlatp, blat2,
                 W1, b12, W2, b22, W3, b32, wop, bo2):
    full = lambda s: pl.BlockSpec(s, lambda i: tuple(0 for _ in s))
    return pl.pallas_call(
        _finish_body,
        grid=(NG,),
        in_specs=[
            pl.BlockSpec((2, BN, HID), lambda i: (0, i, 0)),
            pl.BlockSpec((BN, HID), lambda i: (i, 0)),
            pl.BlockSpec((1, 1, BN), lambda i: (i, 0, 0)),
            full((HID, HID)),
            full((HID, HID)),
            full((1, HID)),
            full((NB, 16)),
            full((16, HID)),
            full((1, HID)),
            full((2 * HID, 3 * HID)),
            full((1, 3 * HID)),
            full((3 * HID, 2 * HID)),
            full((1, 2 * HID)),
            full((2 * HID, HID)),
            full((1, HID)),
            full((HID, HID)),
            full((1, HID)),
        ],
        out_specs=pl.BlockSpec((NB, HID), lambda i: (0, 0)),
        out_shape=jax.ShapeDtypeStruct((NB, HID), jnp.float32),
        scratch_shapes=[
            pltpu.VMEM((NB, HID), jnp.float32),
            pltpu.VMEM((NB, HID), jnp.float32),
        ],
    )(acc3, x, batch3, W_src, W_lin, bp2, latp, wlatp, blat2,
      W1, b12, W2, b22, W3, b32, wop, bo2)


def kernel(x, pos, edge_index, lattice, batch, W_lin, W_src, W_dst, W_pos,
           b_pos, W_lat, b_lat, W1, b1, W2, b2, W3, b3, W_out, b_out):
    posp = jnp.pad(pos, ((0, 0), (0, 5)))
    wposp = jnp.pad(W_pos, ((0, 5), (0, 0)))
    bp2 = b_pos.reshape(1, HID)

    pdt, qvt = _prep_call(x, posp, W_lin, W_src, wposp, bp2)

    acc = _edge_call()(edge_index[0], edge_index[1],
                       pdt, qvt.reshape(2 * N, HID))

    latp = jnp.pad(lattice, ((0, 0), (0, 7)))
    wlatp = jnp.pad(W_lat, ((0, 7), (0, 0)))
    wop = jnp.pad(W_out, ((0, 0), (0, HID - 1)))
    bo2 = jnp.pad(b_out, (0, HID - 1)).reshape(1, HID)

    out = _finish_call(acc.reshape(2, NR, HID), x,
                       batch.reshape(NG, 1, BN),
                       W_src, W_lin, bp2,
                       latp, wlatp, b_lat.reshape(1, HID),
                       W1, b1.reshape(1, 3 * HID),
                       W2, b2.reshape(1, 2 * HID),
                       W3, b3.reshape(1, HID),
                       wop, bo2)
    return out[:, :1]


# speedup vs baseline: 25.9612x; 4.0650x over previous
import functools

import jax
import jax.numpy as jnp
from jax import lax
from jax.experimental import pallas as pl
from jax.experimental.pallas import tpu as pltpu
from jax.experimental.pallas import tpu_sc as plsc

N = 10000
E = 320000
HID = 128
NB = 64
HH = HID // 2

NR = 10240
TRASH = N
NTILES = 16
K = 80
IB = 800
CHPB = IB // K
EPT = E // NTILES
NBLK = EPT // IB
RPT = NR // NTILES
BN = 1000
NG = N // BN
ER = 2500
BE = 250
NEG = ER // BE


def _prep_body(x_ref, p_ref, wl_ref, ws_ref, wp_ref, bp_ref, pdt_ref, uwt_ref):
    xb = x_ref[...]
    pb = p_ref[...]
    P = jnp.dot(pb, wp_ref[...], preferred_element_type=jnp.float32)
    x_lin = jnp.dot(xb, wl_ref[...], preferred_element_type=jnp.float32)
    a_src = jnp.dot(xb, ws_ref[...], preferred_element_type=jnp.float32)
    bp = bp_ref[...]
    pdt_ref[...] = P + bp
    u = jnp.exp(-(P + a_src))
    w = u * (x_lin - P)
    uwt_ref[0, :, :] = jnp.concatenate([u[:, :HH], w[:, :HH]], axis=1)
    uwt_ref[1, :, :] = jnp.concatenate([u[:, HH:], w[:, HH:]], axis=1)


def _prep_call(x, posp, W_lin, W_src, wposp, bp2):
    full = lambda s: pl.BlockSpec(s, lambda i: (0, 0))
    return pl.pallas_call(
        _prep_body,
        grid=(NG,),
        in_specs=[
            pl.BlockSpec((BN, HID), lambda i: (i, 0)),
            pl.BlockSpec((BN, 8), lambda i: (i, 0)),
            full((HID, HID)),
            full((HID, HID)),
            full((8, HID)),
            full((1, HID)),
        ],
        out_specs=[
            pl.BlockSpec((BN, HID), lambda i: (i, 0)),
            pl.BlockSpec((2, BN, HID), lambda i: (0, i, 0)),
        ],
        out_shape=[
            jax.ShapeDtypeStruct((N, HID), jnp.float32),
            jax.ShapeDtypeStruct((2, N, HID), jnp.float32),
        ],
    )(x, posp, W_lin, W_src, wposp, bp2)


def _idx_body(s_ref, d_ref, srcg_ref, dsts_ref):
    s = s_ref[...]
    d = d_ref[...]
    srcg_ref[0, :, :] = s
    srcg_ref[1, :, :] = s + N
    dsts_ref[...] = jnp.where(s == d, jnp.int32(TRASH), d)


def _idx_call(src2, dst2):
    return pl.pallas_call(
        _idx_body,
        out_shape=[
            jax.ShapeDtypeStruct((2, ER, 128), jnp.int32),
            jax.ShapeDtypeStruct((ER, 128), jnp.int32),
        ],
    )(src2, dst2)


def _edge_body(srcg_hbm, dsts_hbm, uwt_hbm, out_hbm,
               v0, v1, sb0, sb1, db0, db1,
               acc, sem0, sem1, bs0, bs1):
    h = lax.axis_index("c")
    sid = lax.axis_index("s")
    V = (v0, v1)
    SB = (sb0, sb1)
    DB = (db0, db1)
    SEM = (sem0, sem1)
    BS = (bs0, bs1)

    def zbody(j, _):
        for q in range(HID // 16):
            v0[j, pl.ds(q * 16, 16)] = jnp.zeros((16,), jnp.float32)
        return 0
    lax.fori_loop(0, K, zbody, 0)
    for k in range(RPT // K):
        pltpu.sync_copy(v0, acc.at[pl.ds(sid * RPT + k * K, K)])
    rem = RPT - (RPT // K) * K
    if rem:
        pltpu.sync_copy(v0.at[pl.ds(0, rem)],
                        acc.at[pl.ds(sid * RPT + (RPT // K) * K, rem)])
    plsc.subcore_barrier()

    ebase = sid * EPT

    def load_blk(i, b):
        base = ebase + i * IB
        cs = pltpu.async_copy(srcg_hbm.at[pl.ds(h * E + base, IB)],
                              SB[b], BS[b])
        cd = pltpu.async_copy(dsts_hbm.at[pl.ds(base, IB)], DB[b], BS[b])
        return cs, cd

    def gather(bb, off, b):
        return pltpu.async_copy(uwt_hbm.at[SB[bb].at[pl.ds(off, K)]],
                                V[b], SEM[b])

    def scatter(bb, off, b):
        pltpu.sync_copy(V[b], acc.at[DB[bb].at[pl.ds(off, K)]], add=True)

    cb = load_blk(0, 0)
    for blk in range(NBLK):
        bb = blk % 2
        cb[0].wait()
        cb[1].wait()
        if blk + 1 < NBLK:
            cb = load_blk(blk + 1, 1 - bb)

        def pair(p, _):
            off0 = (2 * p) * K
            off1 = off0 + K
            cp0 = gather(bb, off0, 0)
            cp1 = gather(bb, off1, 1)
            cp0.wait()
            scatter(bb, off0, 0)
            cp1.wait()
            scatter(bb, off1, 1)
            return 0
        lax.fori_loop(0, CHPB // 2, pair, 0)

    plsc.subcore_barrier()
    pltpu.sync_copy(acc.at[pl.ds(sid * RPT, RPT)],
                    out_hbm.at[pl.ds(h * NR + sid * RPT, RPT)])


@functools.cache
def _edge_call():
    return pl.kernel(
        _edge_body,
        out_type=jax.ShapeDtypeStruct((2 * NR, HID), jnp.float32),
        mesh=plsc.VectorSubcoreMesh(core_axis_name="c", subcore_axis_name="s"),
        scratch_types=[
            pltpu.VMEM((K, HID), jnp.float32),
            pltpu.VMEM((K, HID), jnp.float32),
            pltpu.VMEM((IB,), jnp.int32),
            pltpu.VMEM((IB,), jnp.int32),
            pltpu.VMEM((IB,), jnp.int32),
            pltpu.VMEM((IB,), jnp.int32),
            pltpu.VMEM_SHARED((NR, HID), jnp.float32),
            pltpu.SemaphoreType.DMA,
            pltpu.SemaphoreType.DMA,
            pltpu.SemaphoreType.DMA,
            pltpu.SemaphoreType.DMA,
        ],
    )


def _finish_body(acc_ref, x_ref, pdt_ref, b_ref, ws_ref, wl_ref, bp_ref,
                 lat_ref, wlat_ref, blat_ref,
                 w1_ref, b1_ref, w2_ref, b2_ref, w3_ref, b3_ref,
                 wo_ref, bo_ref, out_ref, pooled_ref, cnt_ref):
    i = pl.program_id(0)
    xb = x_ref[...]
    a_src = jnp.dot(xb, ws_ref[...], preferred_element_type=jnp.float32)
    x_lin = jnp.dot(xb, wl_ref[...], preferred_element_type=jnp.float32)
    bp = bp_ref[...]
    pd = pdt_ref[...]
    epd = jnp.exp(pd)
    es = jnp.exp(bp - a_src)
    a0 = acc_ref[0, :, :]
    a1 = acc_ref[1, :, :]
    U = jnp.concatenate([a0[:, :HH], a1[:, :HH]], axis=1)
    Wn = jnp.concatenate([a0[:, HH:], a1[:, HH:]], axis=1)
    den = epd * U + es
    num = epd * (Wn + pd * U) + es * (x_lin + bp)
    nodes = num / (den + 1e-16)

    b = b_ref[...].reshape(1, BN)
    onehot = (b == lax.broadcasted_iota(jnp.int32, (NB, BN), 0))
    onehot = onehot.astype(jnp.float32)
    ps = jnp.dot(onehot, nodes, preferred_element_type=jnp.float32)
    cs = jnp.dot(onehot, jnp.ones((BN, HID), jnp.float32),
                 preferred_element_type=jnp.float32)

    @pl.when(i == 0)
    def _():
        pooled_ref[...] = ps
        cnt_ref[...] = cs

    @pl.when(i > 0)
    def _():
        pooled_ref[...] += ps
        cnt_ref[...] += cs

    @pl.when(i == NG - 1)
    def _():
        pooled = pooled_ref[...] / jnp.maximum(cnt_ref[...], 1.0)
        l = jnp.dot(lat_ref[...], wlat_ref[...],
                    preferred_element_type=jnp.float32) + blat_ref[...]
        hh = jnp.concatenate([pooled, l], axis=1)
        hh = jnp.maximum(jnp.dot(hh, w1_ref[...],
                                 preferred_element_type=jnp.float32)
                         + b1_ref[...], 0.0)
        hh = jnp.maximum(jnp.dot(hh, w2_ref[...],
                                 preferred_element_type=jnp.float32)
                         + b2_ref[...], 0.0)
        hh = jnp.maximum(jnp.dot(hh, w3_ref[...],
                                 preferred_element_type=jnp.float32)
                         + b3_ref[...], 0.0)
        out_ref[...] = jnp.dot(hh, wo_ref[...],
                               preferred_element_type=jnp.float32) + bo_ref[...]


def _finish_call(acc3, x, pdt, batch3, W_src, W_lin, bp2, latp, wlatp, blat2,
                 W1, b12, W2, b22, W3, b32, wop, bo2):
    full = lambda s: pl.BlockSpec(s, lambda i: tuple(0 for _ in s))
    return pl.pallas_call(
        _finish_body,
        grid=(NG,),
        in_specs=[
            pl.BlockSpec((2, BN, HID), lambda i: (0, i, 0)),
            pl.BlockSpec((BN, HID), lambda i: (i, 0)),
            pl.BlockSpec((BN, HID), lambda i: (i, 0)),
            pl.BlockSpec((1, 1, BN), lambda i: (i, 0, 0)),
            full((HID, HID)),
            full((HID, HID)),
            full((1, HID)),
            full((NB, 16)),
            full((16, HID)),
            full((1, HID)),
            full((2 * HID, 3 * HID)),
            full((1, 3 * HID)),
            full((3 * HID, 2 * HID)),
            full((1, 2 * HID)),
            full((2 * HID, HID)),
            full((1, HID)),
            full((HID, HID)),
            full((1, HID)),
        ],
        out_specs=pl.BlockSpec((NB, HID), lambda i: (0, 0)),
        out_shape=jax.ShapeDtypeStruct((NB, HID), jnp.float32),
        scratch_shapes=[
            pltpu.VMEM((NB, HID), jnp.float32),
            pltpu.VMEM((NB, HID), jnp.float32),
        ],
    )(acc3, x, pdt, batch3, W_src, W_lin, bp2, latp, wlatp, blat2,
      W1, b12, W2, b22, W3, b32, wop, bo2)


def kernel(x, pos, edge_index, lattice, batch, W_lin, W_src, W_dst, W_pos,
           b_pos, W_lat, b_lat, W1, b1, W2, b2, W3, b3, W_out, b_out):
    posp = jnp.pad(pos, ((0, 0), (0, 5)))
    wposp = jnp.pad(W_pos, ((0, 5), (0, 0)))
    bp2 = b_pos.reshape(1, HID)

    pdt, uwt = _prep_call(x, posp, W_lin, W_src, wposp, bp2)
    srcg, dsts = _idx_call(edge_index[0].reshape(ER, 128),
                           edge_index[1].reshape(ER, 128))

    acc = _edge_call()(srcg.reshape(2 * E), dsts.reshape(E),
                       uwt.reshape(2 * N, HID))

    latp = jnp.pad(lattice, ((0, 0), (0, 7)))
    wlatp = jnp.pad(W_lat, ((0, 7), (0, 0)))
    wop = jnp.pad(W_out, ((0, 0), (0, HID - 1)))
    bo2 = jnp.pad(b_out, (0, HID - 1)).reshape(1, HID)

    out = _finish_call(acc.reshape(2, NR, HID), x, pdt,
                       batch.reshape(NG, 1, BN),
                       W_src, W_lin, bp2,
                       latp, wlatp, b_lat.reshape(1, HID),
                       W1, b1.reshape(1, 3 * HID),
                       W2, b2.reshape(1, 2 * HID),
                       W3, b3.reshape(1, HID),
                       wop, bo2)
    return out[:, :1]


# speedup vs baseline: 32.6018x; 1.2558x over previous
import functools

import jax
import jax.numpy as jnp
from jax import lax
from jax.experimental import pallas as pl
from jax.experimental.pallas import tpu as pltpu
from jax.experimental.pallas import tpu_sc as plsc

N = 10000
E = 320000
HID = 128
NB = 64
HH = HID // 2

NR = 10240
TRASH = N
NTILES = 16
K = 80
EPT = E // NTILES
IB = 2000
NIB = EPT // IB
NCB = IB // K
RPT = NR // NTILES
BN = 1000
NG = N // BN
ER = 2500
BE = 250
NEG = ER // BE


def _prep_body(x_ref, p_ref, wl_ref, ws_ref, wp_ref, bp_ref, pdt_ref, uwt_ref):
    xb = x_ref[...]
    pb = p_ref[...]
    P = jnp.dot(pb, wp_ref[...], preferred_element_type=jnp.float32)
    x_lin = jnp.dot(xb, wl_ref[...], preferred_element_type=jnp.float32)
    a_src = jnp.dot(xb, ws_ref[...], preferred_element_type=jnp.float32)
    bp = bp_ref[...]
    pdt_ref[...] = P + bp
    u = jnp.exp(-(P + a_src))
    w = u * (x_lin - P)
    uwt_ref[0, :, :] = jnp.concatenate([u[:, :HH], w[:, :HH]], axis=1)
    uwt_ref[1, :, :] = jnp.concatenate([u[:, HH:], w[:, HH:]], axis=1)


def _prep_call(x, posp, W_lin, W_src, wposp, bp2):
    full = lambda s: pl.BlockSpec(s, lambda i: (0, 0))
    return pl.pallas_call(
        _prep_body,
        grid=(NG,),
        in_specs=[
            pl.BlockSpec((BN, HID), lambda i: (i, 0)),
            pl.BlockSpec((BN, 8), lambda i: (i, 0)),
            full((HID, HID)),
            full((HID, HID)),
            full((8, HID)),
            full((1, HID)),
        ],
        out_specs=[
            pl.BlockSpec((BN, HID), lambda i: (i, 0)),
            pl.BlockSpec((2, BN, HID), lambda i: (0, i, 0)),
        ],
        out_shape=[
            jax.ShapeDtypeStruct((N, HID), jnp.float32),
            jax.ShapeDtypeStruct((2, N, HID), jnp.float32),
        ],
    )(x, posp, W_lin, W_src, wposp, bp2)


def _idx_body(s_ref, d_ref, srcg_ref, dsts_ref):
    s = s_ref[...]
    d = d_ref[...]
    srcg_ref[0, :, :] = s
    srcg_ref[1, :, :] = s + N
    dsts_ref[...] = jnp.where(s == d, jnp.int32(TRASH), d)


def _idx_call(src2, dst2):
    return pl.pallas_call(
        _idx_body,
        out_shape=[
            jax.ShapeDtypeStruct((2, ER, 128), jnp.int32),
            jax.ShapeDtypeStruct((ER, 128), jnp.int32),
        ],
    )(src2, dst2)


def _edge_body(srcg_hbm, dsts_hbm, uwt_hbm, out_hbm,
               v0, v1, sidx, didx,
               acc, sem0, sem1, isem, bsem):
    h = lax.axis_index("c")
    sid = lax.axis_index("s")
    V = (v0, v1)
    SEM = (sem0, sem1)
    ebase = sid * EPT

    def iload(blk):
        par = blk % 2
        cs = pltpu.async_copy(
            srcg_hbm.at[pl.ds(h * E + ebase + blk * IB, IB)],
            sidx.at[pl.ds(par * IB, IB)], isem)
        cd = pltpu.async_copy(
            dsts_hbm.at[pl.ds(ebase + blk * IB, IB)],
            didx.at[pl.ds(par * IB, IB)], bsem)
        return cs, cd

    handles = iload(0)

    def zbody(j, _):
        for q in range(HID // 16):
            v0[j, pl.ds(q * 16, 16)] = jnp.zeros((16,), jnp.float32)
        return 0
    lax.fori_loop(0, K, zbody, 0)
    for k in range(RPT // K):
        pltpu.sync_copy(v0, acc.at[pl.ds(sid * RPT + k * K, K)])
    plsc.subcore_barrier()

    t = 0
    for blk in range(NIB):
        off = (blk % 2) * IB
        cs, cd = handles
        cs.wait()
        cd.wait()
        if blk + 1 < NIB:
            handles = iload(blk + 1)
        gh = pltpu.async_copy(
            uwt_hbm.at[sidx.at[pl.ds(off, K)]], V[t % 2], SEM[t % 2])
        for j in range(NCB):
            b = t % 2
            if j + 1 < NCB:
                nb = (t + 1) % 2
                gh_next = pltpu.async_copy(
                    uwt_hbm.at[sidx.at[pl.ds(off + (j + 1) * K, K)]],
                    V[nb], SEM[nb])
            gh.wait()
            pltpu.sync_copy(V[b], acc.at[didx.at[pl.ds(off + j * K, K)]],
                            add=True)
            if j + 1 < NCB:
                gh = gh_next
            t += 1

    plsc.subcore_barrier()
    pltpu.sync_copy(acc.at[pl.ds(sid * RPT, RPT)],
                    out_hbm.at[pl.ds(h * NR + sid * RPT, RPT)])


@functools.cache
def _edge_call():
    return pl.kernel(
        _edge_body,
        out_type=jax.ShapeDtypeStruct((2 * NR, HID), jnp.float32),
        mesh=plsc.VectorSubcoreMesh(core_axis_name="c", subcore_axis_name="s"),
        scratch_types=[
            pltpu.VMEM((K, HID), jnp.float32),
            pltpu.VMEM((K, HID), jnp.float32),
            pltpu.VMEM((2 * IB,), jnp.int32),
            pltpu.VMEM((2 * IB,), jnp.int32),
            pltpu.VMEM_SHARED((NR, HID), jnp.float32),
            pltpu.SemaphoreType.DMA,
            pltpu.SemaphoreType.DMA,
            pltpu.SemaphoreType.DMA,
            pltpu.SemaphoreType.DMA,
        ],
    )


def _finish_body(acc_ref, x_ref, pdt_ref, b_ref, ws_ref, wl_ref, bp_ref,
                 lat_ref, wlat_ref, blat_ref,
                 w1_ref, b1_ref, w2_ref, b2_ref, w3_ref, b3_ref,
                 wo_ref, bo_ref, out_ref, pooled_ref, cnt_ref):
    i = pl.program_id(0)
    xb = x_ref[...]
    a_src = jnp.dot(xb, ws_ref[...], preferred_element_type=jnp.float32)
    x_lin = jnp.dot(xb, wl_ref[...], preferred_element_type=jnp.float32)
    bp = bp_ref[...]
    pd = pdt_ref[...]
    epd = jnp.exp(pd)
    es = jnp.exp(bp - a_src)
    a0 = acc_ref[0, :, :]
    a1 = acc_ref[1, :, :]
    U = jnp.concatenate([a0[:, :HH], a1[:, :HH]], axis=1)
    Wn = jnp.concatenate([a0[:, HH:], a1[:, HH:]], axis=1)
    den = epd * U + es
    num = epd * (Wn + pd * U) + es * (x_lin + bp)
    nodes = num / (den + 1e-16)

    b = b_ref[...].reshape(1, BN)
    onehot = (b == lax.broadcasted_iota(jnp.int32, (NB, BN), 0))
    onehot = onehot.astype(jnp.float32)
    ps = jnp.dot(onehot, nodes, preferred_element_type=jnp.float32)
    cs = jnp.dot(onehot, jnp.ones((BN, HID), jnp.float32),
                 preferred_element_type=jnp.float32)

    @pl.when(i == 0)
    def _():
        pooled_ref[...] = ps
        cnt_ref[...] = cs

    @pl.when(i > 0)
    def _():
        pooled_ref[...] += ps
        cnt_ref[...] += cs

    @pl.when(i == NG - 1)
    def _():
        pooled = pooled_ref[...] / jnp.maximum(cnt_ref[...], 1.0)
        l = jnp.dot(lat_ref[...], wlat_ref[...],
                    preferred_element_type=jnp.float32) + blat_ref[...]
        hh = jnp.concatenate([pooled, l], axis=1)
        hh = jnp.maximum(jnp.dot(hh, w1_ref[...],
                                 preferred_element_type=jnp.float32)
                         + b1_ref[...], 0.0)
        hh = jnp.maximum(jnp.dot(hh, w2_ref[...],
                                 preferred_element_type=jnp.float32)
                         + b2_ref[...], 0.0)
        hh = jnp.maximum(jnp.dot(hh, w3_ref[...],
                                 preferred_element_type=jnp.float32)
                         + b3_ref[...], 0.0)
        out_ref[...] = jnp.dot(hh, wo_ref[...],
                               preferred_element_type=jnp.float32) + bo_ref[...]


def _finish_call(acc3, x, pdt, batch3, W_src, W_lin, bp2, latp, wlatp, blat2,
                 W1, b12, W2, b22, W3, b32, wop, bo2):
    full = lambda s: pl.BlockSpec(s, lambda i: tuple(0 for _ in s))
    return pl.pallas_call(
        _finish_body,
        grid=(NG,),
        in_specs=[
            pl.BlockSpec((2, BN, HID), lambda i: (0, i, 0)),
            pl.BlockSpec((BN, HID), lambda i: (i, 0)),
            pl.BlockSpec((BN, HID), lambda i: (i, 0)),
            pl.BlockSpec((1, 1, BN), lambda i: (i, 0, 0)),
            full((HID, HID)),
            full((HID, HID)),
            full((1, HID)),
            full((NB, 16)),
            full((16, HID)),
            full((1, HID)),
            full((2 * HID, 3 * HID)),
            full((1, 3 * HID)),
            full((3 * HID, 2 * HID)),
            full((1, 2 * HID)),
            full((2 * HID, HID)),
            full((1, HID)),
            full((HID, HID)),
            full((1, HID)),
        ],
        out_specs=pl.BlockSpec((NB, HID), lambda i: (0, 0)),
        out_shape=jax.ShapeDtypeStruct((NB, HID), jnp.float32),
        scratch_shapes=[
            pltpu.VMEM((NB, HID), jnp.float32),
            pltpu.VMEM((NB, HID), jnp.float32),
        ],
    )(acc3, x, pdt, batch3, W_src, W_lin, bp2, latp, wlatp, blat2,
      W1, b12, W2, b22, W3, b32, wop, bo2)


def kernel(x, pos, edge_index, lattice, batch, W_lin, W_src, W_dst, W_pos,
           b_pos, W_lat, b_lat, W1, b1, W2, b2, W3, b3, W_out, b_out):
    posp = jnp.pad(pos, ((0, 0), (0, 5)))
    wposp = jnp.pad(W_pos, ((0, 5), (0, 0)))
    bp2 = b_pos.reshape(1, HID)

    pdt, uwt = _prep_call(x, posp, W_lin, W_src, wposp, bp2)
    srcg, dsts = _idx_call(edge_index[0].reshape(ER, 128),
                           edge_index[1].reshape(ER, 128))

    acc = _edge_call()(srcg.reshape(2 * E), dsts.reshape(E),
                       uwt.reshape(2 * N, HID))

    latp = jnp.pad(lattice, ((0, 0), (0, 7)))
    wlatp = jnp.pad(W_lat, ((0, 7), (0, 0)))
    wop = jnp.pad(W_out, ((0, 0), (0, HID - 1)))
    bo2 = jnp.pad(b_out, (0, HID - 1)).reshape(1, HID)

    out = _finish_call(acc.reshape(2, NR, HID), x, pdt,
                       batch.reshape(NG, 1, BN),
                       W_src, W_lin, bp2,
                       latp, wlatp, b_lat.reshape(1, HID),
                       W1, b1.reshape(1, 3 * HID),
                       W2, b2.reshape(1, 2 * HID),
                       W3, b3.reshape(1, HID),
                       wop, bo2)
    return out[:, :1]


# speedup vs baseline: 37.5263x; 1.1511x over previous
import functools

import jax
import jax.numpy as jnp
from jax import lax
from jax.experimental import pallas as pl
from jax.experimental.pallas import tpu as pltpu
from jax.experimental.pallas import tpu_sc as plsc

N = 10000
E = 320000
HID = 128
NB = 64
HH = HID // 2

NR = 10240
TRASH = N
NTILES = 16
K = 80
EPT = E // NTILES
IB = 2000
NIB = EPT // IB
NCB = IB // K
NBUF = 3
RPT = NR // NTILES
BN = 1000
NG = N // BN
ER = 2500
BE = 250
NEG = ER // BE


def _prep_body(x_ref, p_ref, wl_ref, ws_ref, wp_ref, bp_ref, pdt_ref, uwt_ref):
    xb = x_ref[...]
    pb = p_ref[...]
    P = jnp.dot(pb, wp_ref[...], preferred_element_type=jnp.float32)
    x_lin = jnp.dot(xb, wl_ref[...], preferred_element_type=jnp.float32)
    a_src = jnp.dot(xb, ws_ref[...], preferred_element_type=jnp.float32)
    bp = bp_ref[...]
    pdt_ref[...] = P + bp
    u = jnp.exp(-(P + a_src))
    w = u * (x_lin - P)
    uwt_ref[0, :, :] = jnp.concatenate([u[:, :HH], w[:, :HH]], axis=1)
    uwt_ref[1, :, :] = jnp.concatenate([u[:, HH:], w[:, HH:]], axis=1)


def _prep_call(x, posp, W_lin, W_src, wposp, bp2):
    full = lambda s: pl.BlockSpec(s, lambda i: (0, 0))
    return pl.pallas_call(
        _prep_body,
        grid=(NG,),
        in_specs=[
            pl.BlockSpec((BN, HID), lambda i: (i, 0)),
            pl.BlockSpec((BN, 8), lambda i: (i, 0)),
            full((HID, HID)),
            full((HID, HID)),
            full((8, HID)),
            full((1, HID)),
        ],
        out_specs=[
            pl.BlockSpec((BN, HID), lambda i: (i, 0)),
            pl.BlockSpec((2, BN, HID), lambda i: (0, i, 0)),
        ],
        out_shape=[
            jax.ShapeDtypeStruct((N, HID), jnp.float32),
            jax.ShapeDtypeStruct((2, N, HID), jnp.float32),
        ],
    )(x, posp, W_lin, W_src, wposp, bp2)


def _idx_body(s_ref, d_ref, srcg_ref, dsts_ref):
    s = s_ref[...]
    d = d_ref[...]
    srcg_ref[0, :, :] = s
    srcg_ref[1, :, :] = s + N
    dsts_ref[...] = jnp.where(s == d, jnp.int32(TRASH), d)


def _idx_call(src2, dst2):
    return pl.pallas_call(
        _idx_body,
        out_shape=[
            jax.ShapeDtypeStruct((2, ER, 128), jnp.int32),
            jax.ShapeDtypeStruct((ER, 128), jnp.int32),
        ],
    )(src2, dst2)


def _edge_body(srcg_hbm, dsts_hbm, uwt_hbm, out_hbm,
               v0, v1, v2, sidx, didx,
               acc, gs0, gs1, gs2, ss0, ss1, ss2, isem, bsem):
    h = lax.axis_index("c")
    sid = lax.axis_index("s")
    V = (v0, v1, v2)
    GS = (gs0, gs1, gs2)
    SS = (ss0, ss1, ss2)
    ebase = sid * EPT

    def iload(blk):
        par = blk % 2
        cs = pltpu.async_copy(
            srcg_hbm.at[pl.ds(h * E + ebase + blk * IB, IB)],
            sidx.at[pl.ds(par * IB, IB)], isem)
        cd = pltpu.async_copy(
            dsts_hbm.at[pl.ds(ebase + blk * IB, IB)],
            didx.at[pl.ds(par * IB, IB)], bsem)
        return cs, cd

    handles = iload(0)

    def zbody(j, _):
        for q in range(HID // 16):
            v0[j, pl.ds(q * 16, 16)] = jnp.zeros((16,), jnp.float32)
        return 0
    lax.fori_loop(0, K, zbody, 0)
    for k in range(RPT // K):
        pltpu.sync_copy(v0, acc.at[pl.ds(sid * RPT + k * K, K)])
    plsc.subcore_barrier()

    t = 0
    sc_h = [None] * NBUF
    npre = NBUF - 1
    for blk in range(NIB):
        off = (blk % 2) * IB
        cs, cd = handles
        cs.wait()
        cd.wait()
        if blk + 1 < NIB:
            handles = iload(blk + 1)
        gh = [None] * NCB
        for j in range(npre):
            b = (t + j) % NBUF
            if sc_h[b] is not None:
                sc_h[b].wait()
                sc_h[b] = None
            gh[j] = pltpu.async_copy(
                uwt_hbm.at[sidx.at[pl.ds(off + j * K, K)]], V[b], GS[b])
        for j in range(NCB):
            b = t % NBUF
            gh[j].wait()
            sc_h[b] = pltpu.async_copy(
                V[b], acc.at[didx.at[pl.ds(off + j * K, K)]], SS[b],
                add=True)
            jn = j + npre
            if jn < NCB:
                bn = (t + npre) % NBUF
                if sc_h[bn] is not None:
                    sc_h[bn].wait()
                    sc_h[bn] = None
                gh[jn] = pltpu.async_copy(
                    uwt_hbm.at[sidx.at[pl.ds(off + jn * K, K)]],
                    V[bn], GS[bn])
            t += 1

    for b in range(NBUF):
        if sc_h[b] is not None:
            sc_h[b].wait()
    plsc.subcore_barrier()
    pltpu.sync_copy(acc.at[pl.ds(sid * RPT, RPT)],
                    out_hbm.at[pl.ds(h * NR + sid * RPT, RPT)])


@functools.cache
def _edge_call():
    return pl.kernel(
        _edge_body,
        out_type=jax.ShapeDtypeStruct((2 * NR, HID), jnp.float32),
        mesh=plsc.VectorSubcoreMesh(core_axis_name="c", subcore_axis_name="s"),
        scratch_types=[
            pltpu.VMEM((K, HID), jnp.float32),
            pltpu.VMEM((K, HID), jnp.float32),
            pltpu.VMEM((K, HID), jnp.float32),
            pltpu.VMEM((2 * IB,), jnp.int32),
            pltpu.VMEM((2 * IB,), jnp.int32),
            pltpu.VMEM_SHARED((NR, HID), jnp.float32),
            pltpu.SemaphoreType.DMA,
            pltpu.SemaphoreType.DMA,
            pltpu.SemaphoreType.DMA,
            pltpu.SemaphoreType.DMA,
            pltpu.SemaphoreType.DMA,
            pltpu.SemaphoreType.DMA,
            pltpu.SemaphoreType.DMA,
            pltpu.SemaphoreType.DMA,
        ],
    )


def _finish_body(acc_ref, x_ref, pdt_ref, b_ref, ws_ref, wl_ref, bp_ref,
                 lat_ref, wlat_ref, blat_ref,
                 w1_ref, b1_ref, w2_ref, b2_ref, w3_ref, b3_ref,
                 wo_ref, bo_ref, out_ref, pooled_ref, cnt_ref):
    i = pl.program_id(0)
    xb = x_ref[...]
    a_src = jnp.dot(xb, ws_ref[...], preferred_element_type=jnp.float32)
    x_lin = jnp.dot(xb, wl_ref[...], preferred_element_type=jnp.float32)
    bp = bp_ref[...]
    pd = pdt_ref[...]
    epd = jnp.exp(pd)
    es = jnp.exp(bp - a_src)
    a0 = acc_ref[0, :, :]
    a1 = acc_ref[1, :, :]
    U = jnp.concatenate([a0[:, :HH], a1[:, :HH]], axis=1)
    Wn = jnp.concatenate([a0[:, HH:], a1[:, HH:]], axis=1)
    den = epd * U + es
    num = epd * (Wn + pd * U) + es * (x_lin + bp)
    nodes = num / (den + 1e-16)

    b = b_ref[...].reshape(1, BN)
    onehot = (b == lax.broadcasted_iota(jnp.int32, (NB, BN), 0))
    onehot = onehot.astype(jnp.float32)
    ps = jnp.dot(onehot, nodes, preferred_element_type=jnp.float32)
    cs = jnp.dot(onehot, jnp.ones((BN, HID), jnp.float32),
                 preferred_element_type=jnp.float32)

    @pl.when(i == 0)
    def _():
        pooled_ref[...] = ps
        cnt_ref[...] = cs

    @pl.when(i > 0)
    def _():
        pooled_ref[...] += ps
        cnt_ref[...] += cs

    @pl.when(i == NG - 1)
    def _():
        pooled = pooled_ref[...] / jnp.maximum(cnt_ref[...], 1.0)
        l = jnp.dot(lat_ref[...], wlat_ref[...],
                    preferred_element_type=jnp.float32) + blat_ref[...]
        hh = jnp.concatenate([pooled, l], axis=1)
        hh = jnp.maximum(jnp.dot(hh, w1_ref[...],
                                 preferred_element_type=jnp.float32)
                         + b1_ref[...], 0.0)
        hh = jnp.maximum(jnp.dot(hh, w2_ref[...],
                                 preferred_element_type=jnp.float32)
                         + b2_ref[...], 0.0)
        hh = jnp.maximum(jnp.dot(hh, w3_ref[...],
                                 preferred_element_type=jnp.float32)
                         + b3_ref[...], 0.0)
        out_ref[...] = jnp.dot(hh, wo_ref[...],
                               preferred_element_type=jnp.float32) + bo_ref[...]


def _finish_call(acc3, x, pdt, batch3, W_src, W_lin, bp2, latp, wlatp, blat2,
                 W1, b12, W2, b22, W3, b32, wop, bo2):
    full = lambda s: pl.BlockSpec(s, lambda i: tuple(0 for _ in s))
    return pl.pallas_call(
        _finish_body,
        grid=(NG,),
        in_specs=[
            pl.BlockSpec((2, BN, HID), lambda i: (0, i, 0)),
            pl.BlockSpec((BN, HID), lambda i: (i, 0)),
            pl.BlockSpec((BN, HID), lambda i: (i, 0)),
            pl.BlockSpec((1, 1, BN), lambda i: (i, 0, 0)),
            full((HID, HID)),
            full((HID, HID)),
            full((1, HID)),
            full((NB, 16)),
            full((16, HID)),
            full((1, HID)),
            full((2 * HID, 3 * HID)),
            full((1, 3 * HID)),
            full((3 * HID, 2 * HID)),
            full((1, 2 * HID)),
            full((2 * HID, HID)),
            full((1, HID)),
            full((HID, HID)),
            full((1, HID)),
        ],
        out_specs=pl.BlockSpec((NB, HID), lambda i: (0, 0)),
        out_shape=jax.ShapeDtypeStruct((NB, HID), jnp.float32),
        scratch_shapes=[
            pltpu.VMEM((NB, HID), jnp.float32),
            pltpu.VMEM((NB, HID), jnp.float32),
        ],
    )(acc3, x, pdt, batch3, W_src, W_lin, bp2, latp, wlatp, blat2,
      W1, b12, W2, b22, W3, b32, wop, bo2)


def kernel(x, pos, edge_index, lattice, batch, W_lin, W_src, W_dst, W_pos,
           b_pos, W_lat, b_lat, W1, b1, W2, b2, W3, b3, W_out, b_out):
    posp = jnp.pad(pos, ((0, 0), (0, 5)))
    wposp = jnp.pad(W_pos, ((0, 5), (0, 0)))
    bp2 = b_pos.reshape(1, HID)

    pdt, uwt = _prep_call(x, posp, W_lin, W_src, wposp, bp2)
    srcg, dsts = _idx_call(edge_index[0].reshape(ER, 128),
                           edge_index[1].reshape(ER, 128))

    acc = _edge_call()(srcg.reshape(2 * E), dsts.reshape(E),
                       uwt.reshape(2 * N, HID))

    latp = jnp.pad(lattice, ((0, 0), (0, 7)))
    wlatp = jnp.pad(W_lat, ((0, 7), (0, 0)))
    wop = jnp.pad(W_out, ((0, 0), (0, HID - 1)))
    bo2 = jnp.pad(b_out, (0, HID - 1)).reshape(1, HID)

    out = _finish_call(acc.reshape(2, NR, HID), x, pdt,
                       batch.reshape(NG, 1, BN),
                       W_src, W_lin, bp2,
                       latp, wlatp, b_lat.reshape(1, HID),
                       W1, b1.reshape(1, 3 * HID),
                       W2, b2.reshape(1, 2 * HID),
                       W3, b3.reshape(1, HID),
                       wop, bo2)
    return out[:, :1]
